# Initial kernel scaffold; baseline (speedup 1.0000x reference)
import jax
import jax.numpy as jnp
from jax.experimental import pallas as pl


def kernel(LLM_label, prob):
    raise NotImplementedError("write your pallas kernel here")


# speedup vs baseline: 2.5984x; 2.5984x over previous
import jax
import jax.numpy as jnp
from jax.experimental import pallas as pl
from jax.experimental.pallas import tpu as pltpu

_B, _T, _C = 128, 50, 8192
_ROWS = _B * _T
_BLK = 400
_NBLK = _ROWS // _BLK


def _onehot_body(prob_ref, lab_ref, out_ref):
    labs = lab_ref[0, 0, :]
    col = jax.lax.broadcasted_iota(jnp.int32, (_BLK, _C), 1)
    mask = col == labs[:, None]
    out_ref[...] = jnp.where(mask, prob_ref[0, 0], 0.0)


def kernel(LLM_label, prob):
    flat = LLM_label.reshape(_NBLK, 1, _BLK).astype(jnp.int32)
    prob2 = prob.reshape(1, 1)
    out = pl.pallas_call(
        _onehot_body,
        grid=(_NBLK,),
        in_specs=[
            pl.BlockSpec(memory_space=pltpu.SMEM),
            pl.BlockSpec((1, 1, _BLK), lambda i: (i, 0, 0)),
        ],
        out_specs=pl.BlockSpec((_BLK, _C), lambda i: (i, 0)),
        out_shape=jax.ShapeDtypeStruct((_ROWS, _C), jnp.float32),
    )(prob2, flat)
    return out.reshape(_B, _T, _C)
